# baseline (speedup 1.0000x reference)
import functools

import numpy as np
import jax
import jax.numpy as jnp
from jax.experimental import pallas as pl

_B, _S = 4096, 200
_VOCAB = 100000
_ROT_A = (13, 15, 26, 6)
_ROT_B = (17, 29, 16, 24)
_TINY = np.float32(1.1754943508222875e-38)


def _np_threefry2x32(k0, k1, c0, c1):
    k0 = np.uint32(k0)
    k1 = np.uint32(k1)
    x0 = (np.asarray(c0, np.uint32) + k0).astype(np.uint32)
    x1 = (np.asarray(c1, np.uint32) + k1).astype(np.uint32)
    ks = (k0, k1, np.uint32(k0 ^ k1 ^ np.uint32(0x1BD11BDA)))
    for d in range(5):
        for r in (_ROT_A if d % 2 == 0 else _ROT_B):
            x0 = (x0 + x1).astype(np.uint32)
            x1 = (((x1 << np.uint32(r)) | (x1 >> np.uint32(32 - r))) ^ x0).astype(np.uint32)
        x0 = (x0 + ks[(d + 1) % 3]).astype(np.uint32)
        x1 = (x1 + ks[(d + 2) % 3] + np.uint32(d + 1)).astype(np.uint32)
    return x0, x1


def _derive_keys(seed):
    o0, o1 = _np_threefry2x32(0, seed, np.uint32([0, 0]), np.uint32([0, 1]))
    k_mask = (int(o0[0]), int(o1[0]))
    k2 = (int(o0[1]), int(o1[1]))
    p0, p1 = _np_threefry2x32(k2[0], k2[1], np.uint32([0, 0]), np.uint32([0, 1]))
    k_samp = (int(p0[1]), int(p1[1]))
    return k_mask, k_samp


_K_MASK, _K_SAMP = _derive_keys(42)


def _tf_bits(key, ctr):
    k0 = np.uint32(key[0])
    k1 = np.uint32(key[1])
    ks2 = np.uint32(k0 ^ k1 ^ np.uint32(0x1BD11BDA))
    ks = (k0, k1, ks2)
    x0 = jnp.full_like(ctr, k0)
    x1 = ctr + k1
    for d in range(5):
        for r in (_ROT_A if d % 2 == 0 else _ROT_B):
            x0 = x0 + x1
            x1 = ((x1 << np.uint32(r)) | (x1 >> np.uint32(32 - r))) ^ x0
        x0 = x0 + ks[(d + 1) % 3]
        x1 = x1 + ks[(d + 2) % 3] + np.uint32(d + 1)
    return x0 ^ x1


def _uniform(bits):
    f = jax.lax.bitcast_convert_type(
        (bits >> np.uint32(9)) | np.uint32(0x3F800000), jnp.float32) - 1.0
    return jnp.maximum(_TINY, f + _TINY)


def _body(x_ref, o_ref, *, block_rows):
    i = pl.program_id(0)
    row = (jax.lax.broadcasted_iota(jnp.uint32, (block_rows, _S), 0)
           + np.uint32(block_rows) * i.astype(jnp.uint32))
    col = jax.lax.broadcasted_iota(jnp.uint32, (block_rows, _S), 1)

    f0 = col * np.uint32(2 * _B) + row * np.uint32(2)
    u0 = _uniform(_tf_bits(_K_MASK, f0))
    u1 = _uniform(_tf_bits(_K_MASK, f0 + np.uint32(1)))
    u2 = u0 * u0
    u4 = u2 * u2
    p9 = u4 * u4 * u0
    drop = u1 <= p9

    v = _tf_bits(_K_SAMP, row * np.uint32(_S) + col)
    vi = jax.lax.bitcast_convert_type(v, jnp.int32)
    q = (v.astype(jnp.float32) * np.float32(1.0 / _VOCAB)).astype(jnp.int32)
    r = vi - q * np.int32(_VOCAB)
    r = jnp.where(r < 0, r + np.int32(_VOCAB), r)
    r = jnp.where(r >= _VOCAB, r - np.int32(_VOCAB), r)
    r = jnp.where(r >= _VOCAB, r - np.int32(_VOCAB), r)

    o_ref[...] = jnp.where(drop, r.astype(jnp.float32), x_ref[...])


def kernel(inputs):
    block_rows = 512
    grid = (_B // block_rows,)
    return pl.pallas_call(
        functools.partial(_body, block_rows=block_rows),
        grid=grid,
        in_specs=[pl.BlockSpec((block_rows, _S), lambda i: (i, 0))],
        out_specs=pl.BlockSpec((block_rows, _S), lambda i: (i, 0)),
        out_shape=jax.ShapeDtypeStruct((_B, _S), jnp.float32),
    )(inputs)
